# Initial kernel scaffold; baseline (speedup 1.0000x reference)
import jax
import jax.numpy as jnp
from jax.experimental import pallas as pl


def kernel(x, x_0, edge_index, W, att, bias):
    raise NotImplementedError("write your pallas kernel here")


# speedup vs baseline: 11.2636x; 11.2636x over previous
import functools

import jax
import jax.numpy as jnp
from jax import lax
from jax.experimental import pallas as pl
from jax.experimental.pallas import tpu as pltpu
from jax.experimental.pallas import tpu_sc as plsc

N = 10000
C = 256
K = 16
K1 = K + 1
NEG_SLOPE = 0.2

NC, NS, L = 2, 16, 16
NW = NC * NS
NPAD = 10240
NPW = NPAD // NW
CH16 = NPW // L
SB = 4
ROWS = SB * K
GS = NPW // SB
NBUF = 4
NJ = C // L
BN = 512


def _tc_stage(x_ref, w_ref, a_ref, h_ref, s_ref):
    xb = x_ref[...]
    hb = lax.dot_general(xb, w_ref[...], (((0,), (1,)), ((), ())),
                         preferred_element_type=jnp.float32)
    hb = jnp.maximum(hb, 0.0)
    h_ref[...] = hb
    s_ref[...] = lax.dot_general(hb, a_ref[...], (((1,), (0,)), ((), ())),
                                 preferred_element_type=jnp.float32)


def _compute_h_s(xp, W, att8):
    return pl.pallas_call(
        _tc_stage,
        grid=(NPAD // BN,),
        in_specs=[
            pl.BlockSpec((C, BN), lambda i: (0, i)),
            pl.BlockSpec((C, C), lambda i: (0, 0)),
            pl.BlockSpec((C, 8), lambda i: (0, 0)),
        ],
        out_specs=[
            pl.BlockSpec((BN, C), lambda i: (i, 0)),
            pl.BlockSpec((BN, 8), lambda i: (i, 0)),
        ],
        out_shape=[
            jax.ShapeDtypeStruct((NPAD, C), jnp.float32),
            jax.ShapeDtypeStruct((NPAD, 8), jnp.float32),
        ],
    )(xp, W, att8)


def _sc_body(h_hbm, si_hbm, sj_hbm, gd_hbm, ga0_hbm, ga1_hbm, bias_hbm,
             out_hbm,
             si_v, sj_v, gd_v, ga0_v, ga1_v, alpha_v, bias_v, *rest):
    rows = rest[:NBUF]
    selfs = rest[NBUF:2 * NBUF]
    obs = rest[2 * NBUF:3 * NBUF]
    gsems = rest[3 * NBUF:4 * NBUF]
    ssems = rest[4 * NBUF:5 * NBUF]
    osems = rest[5 * NBUF:6 * NBUF]
    cid = lax.axis_index("c")
    sid = lax.axis_index("s")
    wid = cid * NS + sid
    base = wid * NPW

    pltpu.sync_copy(si_hbm, si_v)
    pltpu.sync_copy(sj_hbm, sj_v)
    pltpu.sync_copy(gd_hbm.at[wid], gd_v)
    pltpu.sync_copy(ga0_hbm.at[wid], ga0_v)
    pltpu.sync_copy(ga1_hbm.at[wid], ga1_v)
    pltpu.sync_copy(bias_hbm, bias_v)

    def gather_start(g, b):
        pltpu.make_async_copy(h_hbm.at[gd_v.at[g]], rows[b], gsems[b]).start()
        pltpu.make_async_copy(h_hbm.at[pl.ds(base + g * SB, SB)], selfs[b],
                              ssems[b]).start()

    def gather_wait(g, b):
        pltpu.make_async_copy(h_hbm.at[gd_v.at[g]], rows[b], gsems[b]).wait()
        pltpu.make_async_copy(h_hbm.at[pl.ds(base + g * SB, SB)], selfs[b],
                              ssems[b]).wait()

    for b in range(NBUF):
        gather_start(b, b)

    def alpha_chunk(c, carry):
        cb = c * (K1 * L)
        vs = []
        m = jnp.full((L,), -3.4e38, jnp.float32)
        for k in range(K1):
            i0 = ga0_v[pl.ds(cb + k * L, L)]
            i1 = ga1_v[pl.ds(cb + k * L, L)]
            lg = plsc.load_gather(si_v, [i1]) + plsc.load_gather(sj_v, [i0])
            lg = jnp.maximum(lg, NEG_SLOPE * lg)
            vs.append(lg)
            m = jnp.maximum(m, lg)
        tot = jnp.zeros((L,), jnp.float32)
        es = []
        for k in range(K1):
            e = jnp.exp(vs[k] - m)
            es.append(e)
            tot = tot + e
        inv = 1.0 / tot
        for k in range(K1):
            alpha_v[pl.ds(cb + k * L, L)] = es[k] * inv
        return carry

    lax.fori_loop(0, CH16, alpha_chunk, 0)

    bias_regs = tuple(bias_v[j * L:(j + 1) * L] for j in range(NJ))

    def outer(go, carry):
        go4 = go * NBUF
        for b in range(NBUF):
            g = go4 + b
            cb = (g // (L // SB)) * (K1 * L)
            lane_b = (g % (L // SB)) * SB
            gather_wait(g, b)

            @pl.when(go > 0)
            def _(g=g, b=b):
                pltpu.make_async_copy(
                    obs[b], out_hbm.at[pl.ds(base + g * SB, SB)],
                    osems[b]).wait()

            for n in range(SB):
                def kbody(k, acc, n=n, b=b):
                    aidx = jnp.zeros((L,), jnp.int32) + (cb + k * L + lane_b + n)
                    a = plsc.load_gather(alpha_v, [aidx])[0]
                    row = n * K + k
                    return tuple(
                        acc[j] + a * rows[b][row, j * L:(j + 1) * L]
                        for j in range(NJ))

                acc = lax.fori_loop(0, K, kbody, bias_regs)
                sidx = jnp.zeros((L,), jnp.int32) + (cb + K * L + lane_b + n)
                sa = plsc.load_gather(alpha_v, [sidx])[0]
                for j in range(NJ):
                    obs[b][n, j * L:(j + 1) * L] = (
                        acc[j] + sa * selfs[b][n, j * L:(j + 1) * L])

            @pl.when(g + NBUF < GS)
            def _(g=g, b=b):
                gather_start(g + NBUF, b)

            pltpu.make_async_copy(
                obs[b], out_hbm.at[pl.ds(base + g * SB, SB)],
                osems[b]).start()
        return carry

    lax.fori_loop(0, GS // NBUF, outer, 0)
    for b in range(NBUF):
        g = GS - NBUF + b
        pltpu.make_async_copy(
            obs[b], out_hbm.at[pl.ds(base + g * SB, SB)],
            osems[b]).wait()


@functools.partial(
    pl.kernel,
    out_type=jax.ShapeDtypeStruct((NPAD, C), jnp.float32),
    mesh=plsc.VectorSubcoreMesh(core_axis_name="c", subcore_axis_name="s"),
    compiler_params=pltpu.CompilerParams(needs_layout_passes=False),
    scratch_types=(
        [
            pltpu.VMEM((NPAD,), jnp.float32),
            pltpu.VMEM((NPAD,), jnp.float32),
            pltpu.VMEM((GS, ROWS), jnp.int32),
            pltpu.VMEM((CH16 * K1 * L,), jnp.int32),
            pltpu.VMEM((CH16 * K1 * L,), jnp.int32),
            pltpu.VMEM((CH16 * K1 * L,), jnp.float32),
            pltpu.VMEM((C,), jnp.float32),
        ]
        + [pltpu.VMEM((ROWS, C), jnp.float32)] * NBUF
        + [pltpu.VMEM((SB, C), jnp.float32)] * NBUF
        + [pltpu.VMEM((SB, C), jnp.float32)] * NBUF
        + [pltpu.SemaphoreType.DMA] * (3 * NBUF)
    ),
)
def _sc_aggregate(h_hbm, si_hbm, sj_hbm, gd_hbm, ga0_hbm, ga1_hbm, bias_hbm,
                  out_hbm, *rest):
    _sc_body(h_hbm, si_hbm, sj_hbm, gd_hbm, ga0_hbm, ga1_hbm, bias_hbm,
             out_hbm, *rest)


def kernel(x, x_0, edge_index, W, att, bias):
    xf = x[0, :, :, 0]
    xp = jnp.pad(xf, ((0, 0), (0, NPAD - N)))
    att_i = att[0, :C, 0, 0]
    att_j = att[0, C:, 0, 0]
    att8 = jnp.zeros((C, 8), jnp.float32).at[:, 0].set(att_i).at[:, 1].set(att_j)

    hT, s8 = _compute_h_s(xp, W, att8)
    s_i = s8[:, 0]
    s_j = s8[:, 1]

    self_idx = jnp.arange(N, dtype=jnp.int32)[:, None]
    idx0 = jnp.concatenate([edge_index[0, 0], self_idx], axis=1)
    idx1 = jnp.concatenate([edge_index[1, 0], self_idx], axis=1)
    idx0p = jnp.pad(idx0, ((0, NPAD - N), (0, 0)))
    idx1p = jnp.pad(idx1, ((0, NPAD - N), (0, 0)))

    glistD = idx0p[:, :K].reshape(NW, GS, ROWS)
    glistA0 = idx0p.reshape(NW, CH16, L, K1).transpose(0, 1, 3, 2).reshape(
        NW, CH16 * K1 * L)
    glistA1 = idx1p.reshape(NW, CH16, L, K1).transpose(0, 1, 3, 2).reshape(
        NW, CH16 * K1 * L)
    bias_row = bias[0, :, 0, 0]

    out_nm = _sc_aggregate(hT, s_i, s_j, glistD, glistA0, glistA1, bias_row)
    return out_nm[:N].T[None, :, :, None]
